# baseline (speedup 1.0000x reference)
import jax
import jax.numpy as jnp
from jax import lax
from jax.experimental import pallas as pl
from jax.experimental.pallas import tpu as pltpu
from jax.experimental.pallas import tpu_sc as plsc

B, L = 4096, 200
D = 64
N_TOK = B * L
NC, NS, LANES = 2, 16, 16
NW = NC * NS
TOK_PER_W = N_TOK // NW
CHUNK = 256
N_CHUNKS = TOK_PER_W // CHUNK
IDX_MINOR = 128
N_IDX_ROWS = CHUNK // IDX_MINOR
ZROWS = 128


def _sc_body(uix_hbm, lix_hbm, traw_hbm, t_ref, l_ref, u_ref, out_ref,
             uix_v, lix_v, tix_v, tbuf_v, lbuf_v, ubuf_v, zbuf_v,
             sem_in0, sem_in1, sem_g, sem_out0, sem_out1):
    wid = lax.axis_index("s") * NC + lax.axis_index("c")
    w_base = wid * TOK_PER_W
    sem_in = (sem_in0, sem_in1)
    sem_out = (sem_out0, sem_out1)

    def zero_row(i, carry):
        for c in range(2 * D // LANES):
            zbuf_v[i, pl.ds(c * LANES, LANES)] = jnp.zeros((LANES,), jnp.float32)
        return carry
    lax.fori_loop(0, ZROWS, zero_row, 0)

    def stage_idx(i, s):
        base = w_base + i * CHUNK
        for j in range(N_IDX_ROWS):
            seg = pl.ds(base + j * IDX_MINOR, IDX_MINOR)
            pltpu.async_copy(uix_hbm.at[seg], uix_v.at[s, j], sem_in[s])
            pltpu.async_copy(lix_hbm.at[seg], lix_v.at[s, j], sem_in[s])
            pltpu.async_copy(traw_hbm.at[seg], tix_v.at[s, j], sem_in[s])

    def drain_idx(s):
        for j in range(N_IDX_ROWS):
            seg = pl.ds(0, IDX_MINOR)
            pltpu.make_async_copy(uix_hbm.at[seg], uix_v.at[s, j], sem_in[s]).wait()
            pltpu.make_async_copy(lix_hbm.at[seg], lix_v.at[s, j], sem_in[s]).wait()
            pltpu.make_async_copy(traw_hbm.at[seg], tix_v.at[s, j], sem_in[s]).wait()

    def remap_time(s):
        for j in range(N_IDX_ROWS):
            for c in range(IDX_MINOR // LANES):
                sl = pl.ds(c * LANES, LANES)
                t_i = tix_v[s, j, sl]
                tix_v[s, j, sl] = lax.rem(t_i + 23, 24) + 1

    def fire_gathers(s):
        handles = []
        for j in range(N_IDX_ROWS):
            rows = pl.ds(j * IDX_MINOR, IDX_MINOR)
            handles.append(pltpu.async_copy(t_ref.at[tix_v.at[s, j]], tbuf_v.at[s, rows], sem_g))
            handles.append(pltpu.async_copy(l_ref.at[lix_v.at[s, j]], lbuf_v.at[s, rows], sem_g))
            handles.append(pltpu.async_copy(u_ref.at[uix_v.at[s, j]], ubuf_v.at[s, rows], sem_g))
        return handles

    def fire_out(i, s):
        base = w_base + i * CHUNK
        rows = pl.ds(base, CHUNK)
        pltpu.async_copy(tbuf_v.at[s], out_ref.at[rows, pl.ds(0 * D, D)], sem_out[s])
        pltpu.async_copy(lbuf_v.at[s], out_ref.at[rows, pl.ds(1 * D, D)], sem_out[s])
        pltpu.async_copy(ubuf_v.at[s], out_ref.at[rows, pl.ds(2 * D, D)], sem_out[s])
        for z in range(CHUNK // ZROWS):
            zrows = pl.ds(base + z * ZROWS, ZROWS)
            pltpu.async_copy(zbuf_v, out_ref.at[zrows, pl.ds(3 * D, 2 * D)], sem_out[s])

    def drain_out(s):
        rows = pl.ds(0, CHUNK)
        pltpu.make_async_copy(tbuf_v.at[s], out_ref.at[rows, pl.ds(0 * D, D)], sem_out[s]).wait()
        pltpu.make_async_copy(lbuf_v.at[s], out_ref.at[rows, pl.ds(1 * D, D)], sem_out[s]).wait()
        pltpu.make_async_copy(ubuf_v.at[s], out_ref.at[rows, pl.ds(2 * D, D)], sem_out[s]).wait()
        for z in range(CHUNK // ZROWS):
            zrows = pl.ds(z * ZROWS, ZROWS)
            pltpu.make_async_copy(zbuf_v, out_ref.at[zrows, pl.ds(3 * D, 2 * D)], sem_out[s]).wait()

    def step(i, s, not_first):
        drain_idx(s)
        remap_time(s)

        @pl.when(not_first)
        def _():
            drain_out(s)

        handles = fire_gathers(s)
        nxt = jnp.minimum(i + 1, N_CHUNKS - 1)
        stage_idx(nxt, 1 - s)
        for h in handles:
            h.wait()
        fire_out(i, s)

    stage_idx(0, 0)

    def pair_body(k, carry):
        step(2 * k, 0, k >= 1)
        step(2 * k + 1, 1, k >= 1)
        return carry
    lax.fori_loop(0, N_CHUNKS // 2, pair_body, 0)

    drain_idx(0)
    drain_out(0)
    drain_out(1)


def _multi_embed(u_idx, l_idx, t_raw, embed_t_w, embed_l_w, embed_u_w):
    fn = pl.kernel(
        _sc_body,
        out_type=jax.ShapeDtypeStruct((N_TOK, 5 * D), jnp.float32),
        mesh=plsc.VectorSubcoreMesh(core_axis_name="c", subcore_axis_name="s"),
        compiler_params=pltpu.CompilerParams(use_tc_tiling_on_sc=False),
        scratch_types=[
            pltpu.VMEM((2, N_IDX_ROWS, IDX_MINOR), jnp.int32),
            pltpu.VMEM((2, N_IDX_ROWS, IDX_MINOR), jnp.int32),
            pltpu.VMEM((2, N_IDX_ROWS, IDX_MINOR), jnp.int32),
            pltpu.VMEM((2, CHUNK, D), jnp.float32),
            pltpu.VMEM((2, CHUNK, D), jnp.float32),
            pltpu.VMEM((2, CHUNK, D), jnp.float32),
            pltpu.VMEM((ZROWS, 2 * D), jnp.float32),
            pltpu.SemaphoreType.DMA,
            pltpu.SemaphoreType.DMA,
            pltpu.SemaphoreType.DMA,
            pltpu.SemaphoreType.DMA,
            pltpu.SemaphoreType.DMA,
        ],
    )
    return fn(u_idx, l_idx, t_raw, embed_t_w, embed_l_w, embed_u_w)


def kernel(trajectories, embed_t_w, embed_l_w, embed_u_w):
    flat = trajectories.reshape(N_TOK, 3)
    u_idx = flat[:, 0]
    l_idx = flat[:, 1]
    t_raw = flat[:, 2]
    out = _multi_embed(u_idx, l_idx, t_raw, embed_t_w, embed_l_w, embed_u_w)
    return out.reshape(B, L, 5 * D)
